# Initial kernel scaffold; baseline (speedup 1.0000x reference)
import jax
import jax.numpy as jnp
from jax.experimental import pallas as pl


def kernel(x, edge_index, mask_vector, enc_token, dec_token, enc0_W, enc0_b, enc0_g, enc0_beta, enc0_a, enc1_W, enc1_b, enc1_g, enc1_beta, enc1_a, dec0_W, dec0_b, dec0_g, dec0_beta, dec0_a, dec1_W, dec1_b, dec1_g, dec1_beta, dec1_a, proj_W, proj_b):
    raise NotImplementedError("write your pallas kernel here")


# speedup vs baseline: 2.3259x; 2.3259x over previous
import jax
import jax.numpy as jnp
from jax.experimental import pallas as pl

N = 10000
D = 128


def _in_body(x_ref, msk_ref, tok_ref, dinv_ref, W_ref, xin_ref, y_ref):
    m = msk_ref[...]
    xin = x_ref[...] * (1.0 - m) + tok_ref[...] * m
    xin_ref[...] = xin
    y_ref[...] = (
        jnp.dot(xin, W_ref[...], preferred_element_type=jnp.float32)
        * dinv_ref[...]
    )


def _bn_prelu(agg, g, be, a):
    m = jnp.mean(agg, axis=0, keepdims=True)
    v = jnp.mean((agg - m) ** 2, axis=0, keepdims=True)
    o = (agg - m) * jax.lax.rsqrt(v + 1e-5) * g + be
    return jnp.where(o >= 0, o, a * o)


def _mid_body(sa_ref, sb_ref, y_ref, xres_ref, dinv_ref, b_ref, g_ref,
              be_ref, a_ref, W_ref, xout_ref, yout_ref):
    dinv = dinv_ref[...]
    agg = dinv * (sa_ref[...] + sb_ref[...] + y_ref[...]) + b_ref[...]
    o = _bn_prelu(agg, g_ref[...], be_ref[...], a_ref[0, 0])
    x1 = o + xres_ref[...]
    xout_ref[...] = x1
    yout_ref[...] = (
        jnp.dot(x1, W_ref[...], preferred_element_type=jnp.float32) * dinv
    )


def _mid_mask_body(sa_ref, sb_ref, y_ref, xres_ref, dinv_ref, b_ref, g_ref,
                   be_ref, a_ref, msk_ref, tok_ref, W_ref, xout_ref, yout_ref):
    dinv = dinv_ref[...]
    agg = dinv * (sa_ref[...] + sb_ref[...] + y_ref[...]) + b_ref[...]
    o = _bn_prelu(agg, g_ref[...], be_ref[...], a_ref[0, 0])
    x1 = o + xres_ref[...]
    m = msk_ref[...]
    xd = x1 * (1.0 - m) + tok_ref[...] * m
    xout_ref[...] = xd
    yout_ref[...] = (
        jnp.dot(xd, W_ref[...], preferred_element_type=jnp.float32) * dinv
    )


def _final_body(sa_ref, sb_ref, y_ref, xres_ref, dinv_ref, b_ref, g_ref,
                be_ref, a_ref, pW_ref, pb_ref, out_ref):
    dinv = dinv_ref[...]
    agg = dinv * (sa_ref[...] + sb_ref[...] + y_ref[...]) + b_ref[...]
    o = _bn_prelu(agg, g_ref[...], be_ref[...], a_ref[0, 0])
    h = o + xres_ref[...]
    logits = jnp.dot(h, pW_ref[...], preferred_element_type=jnp.float32) + pb_ref[...]
    out_ref[...] = jax.nn.sigmoid(logits)


def _two_out():
    return [
        jax.ShapeDtypeStruct((N, D), jnp.float32),
        jax.ShapeDtypeStruct((N, D), jnp.float32),
    ]


_stage_in = pl.pallas_call(_in_body, out_shape=_two_out())
_stage_mid = pl.pallas_call(_mid_body, out_shape=_two_out())
_stage_mid_mask = pl.pallas_call(_mid_mask_body, out_shape=_two_out())
_stage_final = pl.pallas_call(
    _final_body, out_shape=jax.ShapeDtypeStruct((N, D), jnp.float32))


def kernel(x, edge_index, mask_vector, enc_token, dec_token,
           enc0_W, enc0_b, enc0_g, enc0_beta, enc0_a,
           enc1_W, enc1_b, enc1_g, enc1_beta, enc1_a,
           dec0_W, dec0_b, dec0_g, dec0_beta, dec0_a,
           dec1_W, dec1_b, dec1_g, dec1_beta, dec1_a,
           proj_W, proj_b):
    src = edge_index[0]
    dst = edge_index[1]
    deg = jnp.zeros((N,), jnp.float32).at[dst].add(1.0) + 1.0
    dinv = jax.lax.rsqrt(deg)[:, None]

    msk = (mask_vector == 0).astype(jnp.float32)[:, None]
    zeros = jnp.zeros((N, D), jnp.float32)

    def scatter(y):
        return jnp.zeros((N, D), jnp.float32).at[dst].add(y[src])

    def r2(v):
        return v.reshape(1, -1)

    xin, y0 = _stage_in(x, msk, enc_token, dinv, enc0_W)
    s0 = scatter(y0)
    x1, y1 = _stage_mid(s0, zeros, y0, xin, dinv, r2(enc0_b), r2(enc0_g),
                        r2(enc0_beta), enc0_a.reshape(1, 1), enc1_W)
    s1 = scatter(y1)
    x2, y2 = _stage_mid_mask(s1, zeros, y1, x1, dinv, r2(enc1_b), r2(enc1_g),
                             r2(enc1_beta), enc1_a.reshape(1, 1), msk,
                             dec_token, dec0_W)
    s2 = scatter(y2)
    x3, y3 = _stage_mid(s2, zeros, y2, x2, dinv, r2(dec0_b), r2(dec0_g),
                        r2(dec0_beta), dec0_a.reshape(1, 1), dec1_W)
    s3 = scatter(y3)
    out = _stage_final(s3, zeros, y3, x3, dinv, r2(dec1_b), r2(dec1_g),
                       r2(dec1_beta), dec1_a.reshape(1, 1), proj_W,
                       r2(proj_b))
    return out


# speedup vs baseline: 15.6749x; 6.7394x over previous
import functools

import jax
import jax.numpy as jnp
from jax import lax
from jax.experimental import pallas as pl
from jax.experimental.pallas import tpu as pltpu
from jax.experimental.pallas import tpu_sc as plsc

N = 10000
D = 128
E = 320000
NW = 32
NSUB = 16
CH = 128
NCHUNK = 80
EP = NW * NCHUNK * CH
PAD = EP - E
NP = 10240
STRIPE = NP // NSUB


def _in_body(x_ref, msk_ref, tok_ref, dinv_ref, W_ref, xin_ref, y_ref):
    m = msk_ref[...]
    xin = x_ref[...] * (1.0 - m) + tok_ref[...] * m
    xin_ref[...] = xin
    y_ref[...] = (
        jnp.dot(xin, W_ref[...], preferred_element_type=jnp.float32)
        * dinv_ref[...]
    )


def _bn_prelu(agg, g, be, a):
    m = jnp.mean(agg, axis=0, keepdims=True)
    v = jnp.mean((agg - m) ** 2, axis=0, keepdims=True)
    o = (agg - m) * jax.lax.rsqrt(v + 1e-5) * g + be
    return jnp.where(o >= 0, o, a * o)


def _mid_body(sa_ref, sb_ref, y_ref, xres_ref, dinv_ref, b_ref, g_ref,
              be_ref, a_ref, W_ref, xout_ref, yout_ref):
    dinv = dinv_ref[...]
    agg = dinv * (sa_ref[...] + sb_ref[...] + y_ref[...]) + b_ref[...]
    o = _bn_prelu(agg, g_ref[...], be_ref[...], a_ref[0, 0])
    x1 = o + xres_ref[...]
    xout_ref[...] = x1
    yout_ref[...] = (
        jnp.dot(x1, W_ref[...], preferred_element_type=jnp.float32) * dinv
    )


def _mid_mask_body(sa_ref, sb_ref, y_ref, xres_ref, dinv_ref, b_ref, g_ref,
                   be_ref, a_ref, msk_ref, tok_ref, W_ref, xout_ref, yout_ref):
    dinv = dinv_ref[...]
    agg = dinv * (sa_ref[...] + sb_ref[...] + y_ref[...]) + b_ref[...]
    o = _bn_prelu(agg, g_ref[...], be_ref[...], a_ref[0, 0])
    x1 = o + xres_ref[...]
    m = msk_ref[...]
    xd = x1 * (1.0 - m) + tok_ref[...] * m
    xout_ref[...] = xd
    yout_ref[...] = (
        jnp.dot(xd, W_ref[...], preferred_element_type=jnp.float32) * dinv
    )


def _final_body(sa_ref, sb_ref, y_ref, xres_ref, dinv_ref, b_ref, g_ref,
                be_ref, a_ref, pW_ref, pb_ref, out_ref):
    dinv = dinv_ref[...]
    agg = dinv * (sa_ref[...] + sb_ref[...] + y_ref[...]) + b_ref[...]
    o = _bn_prelu(agg, g_ref[...], be_ref[...], a_ref[0, 0])
    h = o + xres_ref[...]
    logits = jnp.dot(h, pW_ref[...], preferred_element_type=jnp.float32) + pb_ref[...]
    out_ref[...] = jax.nn.sigmoid(logits)


def _two_out():
    return [
        jax.ShapeDtypeStruct((N, D), jnp.float32),
        jax.ShapeDtypeStruct((N, D), jnp.float32),
    ]


_stage_in = pl.pallas_call(_in_body, out_shape=_two_out())
_stage_mid = pl.pallas_call(_mid_body, out_shape=_two_out())
_stage_mid_mask = pl.pallas_call(_mid_mask_body, out_shape=_two_out())
_stage_final = pl.pallas_call(
    _final_body, out_shape=jax.ShapeDtypeStruct((N, D), jnp.float32))


_mesh = plsc.VectorSubcoreMesh(core_axis_name="c", subcore_axis_name="s")


def _sc_scatter_body(y_hbm, src_hbm, dst_hbm, zero_hbm, out_hbm,
                     srcv, dstv, rows, acc, sem):
    c = lax.axis_index("c")
    s = lax.axis_index("s")
    w = c * NSUB + s
    pltpu.sync_copy(src_hbm.at[w], srcv)
    pltpu.sync_copy(dst_hbm.at[w], dstv)
    pltpu.sync_copy(zero_hbm.at[pl.ds(s * STRIPE, STRIPE)],
                    acc.at[pl.ds(s * STRIPE, STRIPE)])
    plsc.subcore_barrier()

    def chunk(i, carry):
        pltpu.async_copy(y_hbm.at[srcv.at[i]], rows, sem).wait()
        pltpu.sync_copy(rows, acc.at[dstv.at[i]], add=True)
        return carry

    lax.fori_loop(0, NCHUNK, chunk, 0)
    plsc.subcore_barrier()
    pltpu.sync_copy(acc.at[pl.ds(s * STRIPE, STRIPE)],
                    out_hbm.at[c, pl.ds(s * STRIPE, STRIPE)])


_sc_scatter = pl.kernel(
    _sc_scatter_body,
    out_type=jax.ShapeDtypeStruct((2, NP, D), jnp.float32),
    mesh=_mesh,
    scratch_types=[
        pltpu.VMEM((NCHUNK, CH), jnp.int32),
        pltpu.VMEM((NCHUNK, CH), jnp.int32),
        pltpu.VMEM((CH, D), jnp.float32),
        pltpu.VMEM_SHARED((NP, D), jnp.float32),
        pltpu.SemaphoreType.DMA,
    ],
)


def _sc_deg_body(dst_hbm, out_hbm, dstv, zv, ov, acc):
    c = lax.axis_index("c")
    s = lax.axis_index("s")
    w = c * NSUB + s
    pltpu.sync_copy(dst_hbm.at[w], dstv)
    for k in range(CH // 16):
        zv[pl.ds(k * 16, 16)] = jnp.zeros((16,), jnp.float32)
        ov[pl.ds(k * 16, 16)] = jnp.ones((16,), jnp.float32)
    for k in range(STRIPE // CH):
        pltpu.sync_copy(zv, acc.at[pl.ds(s * STRIPE + k * CH, CH)])
    plsc.subcore_barrier()

    def chunk(i, carry):
        pltpu.sync_copy(ov, acc.at[dstv.at[i]], add=True)
        return carry

    lax.fori_loop(0, NCHUNK, chunk, 0)
    plsc.subcore_barrier()
    for k in range(STRIPE // CH):
        pltpu.sync_copy(acc.at[pl.ds(s * STRIPE + k * CH, CH)],
                        out_hbm.at[c, pl.ds(s * STRIPE + k * CH, CH)])


_sc_deg = pl.kernel(
    _sc_deg_body,
    out_type=jax.ShapeDtypeStruct((2, NP), jnp.float32),
    mesh=_mesh,
    scratch_types=[
        pltpu.VMEM((NCHUNK, CH), jnp.int32),
        pltpu.VMEM((CH,), jnp.float32),
        pltpu.VMEM((CH,), jnp.float32),
        pltpu.VMEM_SHARED((NP,), jnp.float32),
    ],
)


def kernel(x, edge_index, mask_vector, enc_token, dec_token,
           enc0_W, enc0_b, enc0_g, enc0_beta, enc0_a,
           enc1_W, enc1_b, enc1_g, enc1_beta, enc1_a,
           dec0_W, dec0_b, dec0_g, dec0_beta, dec0_a,
           dec1_W, dec1_b, dec1_g, dec1_beta, dec1_a,
           proj_W, proj_b):
    src = edge_index[0]
    dst = edge_index[1]
    pidx = jnp.arange(PAD, dtype=jnp.int32)
    srcp = jnp.concatenate([src, pidx % N]).reshape(NW, NCHUNK, CH)
    dstp = jnp.concatenate([dst, N + (pidx % 16)]).reshape(NW, NCHUNK, CH)

    degp = _sc_deg(dstp)
    deg = degp[0, :N] + degp[1, :N] + 1.0
    dinv = jax.lax.rsqrt(deg)[:, None]

    msk = (mask_vector == 0).astype(jnp.float32)[:, None]
    zero2d = jnp.zeros((NP, D), jnp.float32)

    def scatter(y):
        sp = _sc_scatter(y, srcp, dstp, zero2d)
        return sp[0, :N], sp[1, :N]

    def r2(v):
        return v.reshape(1, -1)

    xin, y0 = _stage_in(x, msk, enc_token, dinv, enc0_W)
    s0a, s0b = scatter(y0)
    x1, y1 = _stage_mid(s0a, s0b, y0, xin, dinv, r2(enc0_b), r2(enc0_g),
                        r2(enc0_beta), enc0_a.reshape(1, 1), enc1_W)
    s1a, s1b = scatter(y1)
    x2, y2 = _stage_mid_mask(s1a, s1b, y1, x1, dinv, r2(enc1_b), r2(enc1_g),
                             r2(enc1_beta), enc1_a.reshape(1, 1), msk,
                             dec_token, dec0_W)
    s2a, s2b = scatter(y2)
    x3, y3 = _stage_mid(s2a, s2b, y2, x2, dinv, r2(dec0_b), r2(dec0_g),
                        r2(dec0_beta), dec0_a.reshape(1, 1), dec1_W)
    s3a, s3b = scatter(y3)
    out = _stage_final(s3a, s3b, y3, x3, dinv, r2(dec1_b), r2(dec1_g),
                       r2(dec1_beta), dec1_a.reshape(1, 1), proj_W,
                       r2(proj_b))
    return out


# speedup vs baseline: 23.2182x; 1.4812x over previous
import functools

import jax
import jax.numpy as jnp
from jax import lax
from jax.experimental import pallas as pl
from jax.experimental.pallas import tpu as pltpu
from jax.experimental.pallas import tpu_sc as plsc

N = 10000
D = 128
E = 320000
NW = 32
NSUB = 16
CH = 128
NCHUNK = 80
EP = NW * NCHUNK * CH
PAD = EP - E
NP = 10240
STRIPE = NP // NSUB


def _in_body(x_ref, msk_ref, tok_ref, dinv_ref, W_ref, xin_ref, y_ref):
    m = msk_ref[...]
    xin = x_ref[...] * (1.0 - m) + tok_ref[...] * m
    xin_ref[...] = xin
    y_ref[...] = (
        jnp.dot(xin, W_ref[...], preferred_element_type=jnp.float32)
        * dinv_ref[...]
    )


def _bn_prelu(agg, g, be, a):
    m = jnp.mean(agg, axis=0, keepdims=True)
    v = jnp.mean((agg - m) ** 2, axis=0, keepdims=True)
    o = (agg - m) * jax.lax.rsqrt(v + 1e-5) * g + be
    return jnp.where(o >= 0, o, a * o)


def _mid_body(sa_ref, sb_ref, y_ref, xres_ref, dinv_ref, b_ref, g_ref,
              be_ref, a_ref, W_ref, xout_ref, yout_ref):
    dinv = dinv_ref[...]
    agg = dinv * (sa_ref[...] + sb_ref[...] + y_ref[...]) + b_ref[...]
    o = _bn_prelu(agg, g_ref[...], be_ref[...], a_ref[0, 0])
    x1 = o + xres_ref[...]
    xout_ref[...] = x1
    yout_ref[...] = (
        jnp.dot(x1, W_ref[...], preferred_element_type=jnp.float32) * dinv
    )


def _mid_mask_body(sa_ref, sb_ref, y_ref, xres_ref, dinv_ref, b_ref, g_ref,
                   be_ref, a_ref, msk_ref, tok_ref, W_ref, xout_ref, yout_ref):
    dinv = dinv_ref[...]
    agg = dinv * (sa_ref[...] + sb_ref[...] + y_ref[...]) + b_ref[...]
    o = _bn_prelu(agg, g_ref[...], be_ref[...], a_ref[0, 0])
    x1 = o + xres_ref[...]
    m = msk_ref[...]
    xd = x1 * (1.0 - m) + tok_ref[...] * m
    xout_ref[...] = xd
    yout_ref[...] = (
        jnp.dot(xd, W_ref[...], preferred_element_type=jnp.float32) * dinv
    )


def _final_body(sa_ref, sb_ref, y_ref, xres_ref, dinv_ref, b_ref, g_ref,
                be_ref, a_ref, pW_ref, pb_ref, out_ref):
    dinv = dinv_ref[...]
    agg = dinv * (sa_ref[...] + sb_ref[...] + y_ref[...]) + b_ref[...]
    o = _bn_prelu(agg, g_ref[...], be_ref[...], a_ref[0, 0])
    h = o + xres_ref[...]
    logits = jnp.dot(h, pW_ref[...], preferred_element_type=jnp.float32) + pb_ref[...]
    out_ref[...] = jax.nn.sigmoid(logits)


def _two_out():
    return [
        jax.ShapeDtypeStruct((N, D), jnp.float32),
        jax.ShapeDtypeStruct((N, D), jnp.float32),
    ]


_stage_in = pl.pallas_call(_in_body, out_shape=_two_out())
_stage_mid = pl.pallas_call(_mid_body, out_shape=_two_out())
_stage_mid_mask = pl.pallas_call(_mid_mask_body, out_shape=_two_out())
_stage_final = pl.pallas_call(
    _final_body, out_shape=jax.ShapeDtypeStruct((N, D), jnp.float32))


_mesh = plsc.VectorSubcoreMesh(core_axis_name="c", subcore_axis_name="s")


BLK = 16
NBLK = NCHUNK // BLK


def _sc_scatter_body(y_hbm, src_hbm, dst_hbm, zero_hbm, out_hbm,
                     s0, s1, d0, d1, r0, r1, acc,
                     ib0, ib1, g0, g1, zsem):
    c = lax.axis_index("c")
    s = lax.axis_index("s")
    w = c * NSUB + s
    sv = (s0, s1)
    dv = (d0, d1)
    rows = (r0, r1)
    gsem = (g0, g1)
    ibsem = (ib0, ib1)

    def load_idx_block(blk):
        bf = blk % 2
        pltpu.async_copy(src_hbm.at[w, pl.ds(blk * BLK, BLK)], sv[bf],
                         ibsem[bf])
        pltpu.async_copy(dst_hbm.at[w, pl.ds(blk * BLK, BLK)], dv[bf],
                         ibsem[bf])

    def wait_idx_block(blk):
        bf = blk % 2
        pltpu.make_async_copy(src_hbm.at[w, pl.ds(blk * BLK, BLK)], sv[bf],
                              ibsem[bf]).wait()
        pltpu.make_async_copy(dst_hbm.at[w, pl.ds(blk * BLK, BLK)], dv[bf],
                              ibsem[bf]).wait()

    def gather(ci):
        pltpu.async_copy(y_hbm.at[sv[(ci // BLK) % 2].at[ci % BLK]],
                         rows[ci % 2], gsem[ci % 2])

    def wait_gather(ci):
        pltpu.make_async_copy(y_hbm.at[sv[(ci // BLK) % 2].at[ci % BLK]],
                              rows[ci % 2], gsem[ci % 2]).wait()

    load_idx_block(0)
    load_idx_block(1)
    pltpu.async_copy(zero_hbm.at[pl.ds(s * STRIPE, STRIPE)],
                     acc.at[pl.ds(s * STRIPE, STRIPE)], zsem)
    wait_idx_block(0)
    gather(0)
    gather(1)
    pltpu.make_async_copy(zero_hbm.at[pl.ds(s * STRIPE, STRIPE)],
                          acc.at[pl.ds(s * STRIPE, STRIPE)], zsem).wait()
    plsc.subcore_barrier()

    for ci in range(NCHUNK):
        blk, kb = ci // BLK, ci % BLK
        wait_gather(ci)
        pltpu.sync_copy(rows[ci % 2], acc.at[dv[blk % 2].at[kb]], add=True)
        if kb == BLK - 1 and blk + 2 < NBLK:
            load_idx_block(blk + 2)
        cj = ci + 2
        if cj < NCHUNK:
            if cj % BLK == 0:
                wait_idx_block(cj // BLK)
            gather(cj)

    plsc.subcore_barrier()
    pltpu.sync_copy(acc.at[pl.ds(s * STRIPE, STRIPE)],
                    out_hbm.at[c, pl.ds(s * STRIPE, STRIPE)])


_sc_scatter = pl.kernel(
    _sc_scatter_body,
    out_type=jax.ShapeDtypeStruct((2, NP, D), jnp.float32),
    mesh=_mesh,
    scratch_types=[
        pltpu.VMEM((BLK, CH), jnp.int32),
        pltpu.VMEM((BLK, CH), jnp.int32),
        pltpu.VMEM((BLK, CH), jnp.int32),
        pltpu.VMEM((BLK, CH), jnp.int32),
        pltpu.VMEM((CH, D), jnp.float32),
        pltpu.VMEM((CH, D), jnp.float32),
        pltpu.VMEM_SHARED((NP, D), jnp.float32),
        pltpu.SemaphoreType.DMA,
        pltpu.SemaphoreType.DMA,
        pltpu.SemaphoreType.DMA,
        pltpu.SemaphoreType.DMA,
        pltpu.SemaphoreType.DMA,
    ],
)


def _sc_deg_body(dst_hbm, out_hbm, dstv, zv, ov, acc):
    c = lax.axis_index("c")
    s = lax.axis_index("s")
    w = c * NSUB + s
    pltpu.sync_copy(dst_hbm.at[w], dstv)
    for k in range(CH // 16):
        zv[pl.ds(k * 16, 16)] = jnp.zeros((16,), jnp.float32)
        ov[pl.ds(k * 16, 16)] = jnp.ones((16,), jnp.float32)
    for k in range(STRIPE // CH):
        pltpu.sync_copy(zv, acc.at[pl.ds(s * STRIPE + k * CH, CH)])
    plsc.subcore_barrier()

    def chunk(i, carry):
        pltpu.sync_copy(ov, acc.at[dstv.at[i]], add=True)
        return carry

    lax.fori_loop(0, NCHUNK, chunk, 0)
    plsc.subcore_barrier()
    for k in range(STRIPE // CH):
        pltpu.sync_copy(acc.at[pl.ds(s * STRIPE + k * CH, CH)],
                        out_hbm.at[c, pl.ds(s * STRIPE + k * CH, CH)])


_sc_deg = pl.kernel(
    _sc_deg_body,
    out_type=jax.ShapeDtypeStruct((2, NP), jnp.float32),
    mesh=_mesh,
    scratch_types=[
        pltpu.VMEM((NCHUNK, CH), jnp.int32),
        pltpu.VMEM((CH,), jnp.float32),
        pltpu.VMEM((CH,), jnp.float32),
        pltpu.VMEM_SHARED((NP,), jnp.float32),
    ],
)


def kernel(x, edge_index, mask_vector, enc_token, dec_token,
           enc0_W, enc0_b, enc0_g, enc0_beta, enc0_a,
           enc1_W, enc1_b, enc1_g, enc1_beta, enc1_a,
           dec0_W, dec0_b, dec0_g, dec0_beta, dec0_a,
           dec1_W, dec1_b, dec1_g, dec1_beta, dec1_a,
           proj_W, proj_b):
    src = edge_index[0]
    dst = edge_index[1]
    pidx = jnp.arange(PAD, dtype=jnp.int32)
    srcp = jnp.concatenate([src, pidx % N]).reshape(NW, NCHUNK, CH)
    dstp = jnp.concatenate([dst, N + (pidx % 16)]).reshape(NW, NCHUNK, CH)

    degp = _sc_deg(dstp)
    deg = degp[0, :N] + degp[1, :N] + 1.0
    dinv = jax.lax.rsqrt(deg)[:, None]

    msk = (mask_vector == 0).astype(jnp.float32)[:, None]
    zero2d = jnp.zeros((NP, D), jnp.float32)

    def scatter(y):
        sp = _sc_scatter(y, srcp, dstp, zero2d)
        return sp[0, :N], sp[1, :N]

    def r2(v):
        return v.reshape(1, -1)

    xin, y0 = _stage_in(x, msk, enc_token, dinv, enc0_W)
    s0a, s0b = scatter(y0)
    x1, y1 = _stage_mid(s0a, s0b, y0, xin, dinv, r2(enc0_b), r2(enc0_g),
                        r2(enc0_beta), enc0_a.reshape(1, 1), enc1_W)
    s1a, s1b = scatter(y1)
    x2, y2 = _stage_mid_mask(s1a, s1b, y1, x1, dinv, r2(enc1_b), r2(enc1_g),
                             r2(enc1_beta), enc1_a.reshape(1, 1), msk,
                             dec_token, dec0_W)
    s2a, s2b = scatter(y2)
    x3, y3 = _stage_mid(s2a, s2b, y2, x2, dinv, r2(dec0_b), r2(dec0_g),
                        r2(dec0_beta), dec0_a.reshape(1, 1), dec1_W)
    s3a, s3b = scatter(y3)
    out = _stage_final(s3a, s3b, y3, x3, dinv, r2(dec1_b), r2(dec1_g),
                       r2(dec1_beta), dec1_a.reshape(1, 1), proj_W,
                       r2(proj_b))
    return out


# speedup vs baseline: 24.3134x; 1.0472x over previous
import functools

import jax
import jax.numpy as jnp
from jax import lax
from jax.experimental import pallas as pl
from jax.experimental.pallas import tpu as pltpu
from jax.experimental.pallas import tpu_sc as plsc

N = 10000
D = 128
E = 320000
NW = 32
NSUB = 16
CH = 128
NCHUNK = 80
EP = NW * NCHUNK * CH
PAD = EP - E
NP = 10240
STRIPE = NP // NSUB


def _in_body(x_ref, msk_ref, tok_ref, dinv_ref, W_ref, xin_ref, y_ref):
    m = msk_ref[...]
    xin = x_ref[...] * (1.0 - m) + tok_ref[...] * m
    xin_ref[...] = xin
    y_ref[...] = (
        jnp.dot(xin, W_ref[...], preferred_element_type=jnp.float32)
        * dinv_ref[...]
    )


def _bn_prelu(agg, g, be, a):
    m = jnp.mean(agg, axis=0, keepdims=True)
    v = jnp.mean((agg - m) ** 2, axis=0, keepdims=True)
    o = (agg - m) * jax.lax.rsqrt(v + 1e-5) * g + be
    return jnp.where(o >= 0, o, a * o)


def _mid_body(sp_ref, y_ref, xres_ref, dinv_ref, b_ref, g_ref,
              be_ref, a_ref, W_ref, xout_ref, yout_ref):
    dinv = dinv_ref[...]
    sp = sp_ref[...]
    agg = dinv * (sp[0, :N] + sp[1, :N] + y_ref[...]) + b_ref[...]
    o = _bn_prelu(agg, g_ref[...], be_ref[...], a_ref[0, 0])
    x1 = o + xres_ref[...]
    xout_ref[...] = x1
    yout_ref[...] = (
        jnp.dot(x1, W_ref[...], preferred_element_type=jnp.float32) * dinv
    )


def _mid_mask_body(sp_ref, y_ref, xres_ref, dinv_ref, b_ref, g_ref,
                   be_ref, a_ref, msk_ref, tok_ref, W_ref, xout_ref, yout_ref):
    dinv = dinv_ref[...]
    sp = sp_ref[...]
    agg = dinv * (sp[0, :N] + sp[1, :N] + y_ref[...]) + b_ref[...]
    o = _bn_prelu(agg, g_ref[...], be_ref[...], a_ref[0, 0])
    x1 = o + xres_ref[...]
    m = msk_ref[...]
    xd = x1 * (1.0 - m) + tok_ref[...] * m
    xout_ref[...] = xd
    yout_ref[...] = (
        jnp.dot(xd, W_ref[...], preferred_element_type=jnp.float32) * dinv
    )


def _final_body(sp_ref, y_ref, xres_ref, dinv_ref, b_ref, g_ref,
                be_ref, a_ref, pW_ref, pb_ref, out_ref):
    dinv = dinv_ref[...]
    sp = sp_ref[...]
    agg = dinv * (sp[0, :N] + sp[1, :N] + y_ref[...]) + b_ref[...]
    o = _bn_prelu(agg, g_ref[...], be_ref[...], a_ref[0, 0])
    h = o + xres_ref[...]
    logits = jnp.dot(h, pW_ref[...], preferred_element_type=jnp.float32) + pb_ref[...]
    out_ref[...] = jax.nn.sigmoid(logits)


def _two_out():
    return [
        jax.ShapeDtypeStruct((N, D), jnp.float32),
        jax.ShapeDtypeStruct((N, D), jnp.float32),
    ]


_stage_in = pl.pallas_call(_in_body, out_shape=_two_out())
_stage_mid = pl.pallas_call(_mid_body, out_shape=_two_out())
_stage_mid_mask = pl.pallas_call(_mid_mask_body, out_shape=_two_out())
_stage_final = pl.pallas_call(
    _final_body, out_shape=jax.ShapeDtypeStruct((N, D), jnp.float32))


_mesh = plsc.VectorSubcoreMesh(core_axis_name="c", subcore_axis_name="s")


BLK = 16
NBLK = NCHUNK // BLK


def _sc_scatter_body(y_hbm, src_hbm, dst_hbm, zero_hbm, out_hbm,
                     s0, s1, d0, d1, r0, r1, acc,
                     ib0, ib1, g0, g1, zsem):
    c = lax.axis_index("c")
    s = lax.axis_index("s")
    w = c * NSUB + s
    sv = (s0, s1)
    dv = (d0, d1)
    rows = (r0, r1)
    gsem = (g0, g1)
    ibsem = (ib0, ib1)

    def load_idx_block(blk):
        bf = blk % 2
        pltpu.async_copy(src_hbm.at[w, pl.ds(blk * BLK, BLK)], sv[bf],
                         ibsem[bf])
        pltpu.async_copy(dst_hbm.at[w, pl.ds(blk * BLK, BLK)], dv[bf],
                         ibsem[bf])

    def wait_idx_block(blk):
        bf = blk % 2
        pltpu.make_async_copy(src_hbm.at[w, pl.ds(blk * BLK, BLK)], sv[bf],
                              ibsem[bf]).wait()
        pltpu.make_async_copy(dst_hbm.at[w, pl.ds(blk * BLK, BLK)], dv[bf],
                              ibsem[bf]).wait()

    def gather(ci):
        pltpu.async_copy(y_hbm.at[sv[(ci // BLK) % 2].at[ci % BLK]],
                         rows[ci % 2], gsem[ci % 2])

    def wait_gather(ci):
        pltpu.make_async_copy(y_hbm.at[sv[(ci // BLK) % 2].at[ci % BLK]],
                              rows[ci % 2], gsem[ci % 2]).wait()

    load_idx_block(0)
    load_idx_block(1)
    pltpu.async_copy(zero_hbm.at[pl.ds(s * STRIPE, STRIPE)],
                     acc.at[pl.ds(s * STRIPE, STRIPE)], zsem)
    wait_idx_block(0)
    gather(0)
    gather(1)
    pltpu.make_async_copy(zero_hbm.at[pl.ds(s * STRIPE, STRIPE)],
                          acc.at[pl.ds(s * STRIPE, STRIPE)], zsem).wait()
    plsc.subcore_barrier()

    for ci in range(NCHUNK):
        blk, kb = ci // BLK, ci % BLK
        wait_gather(ci)
        pltpu.sync_copy(rows[ci % 2], acc.at[dv[blk % 2].at[kb]], add=True)
        if kb == BLK - 1 and blk + 2 < NBLK:
            load_idx_block(blk + 2)
        cj = ci + 2
        if cj < NCHUNK:
            if cj % BLK == 0:
                wait_idx_block(cj // BLK)
            gather(cj)

    plsc.subcore_barrier()
    pltpu.sync_copy(acc.at[pl.ds(s * STRIPE, STRIPE)],
                    out_hbm.at[c, pl.ds(s * STRIPE, STRIPE)])


_sc_scatter = pl.kernel(
    _sc_scatter_body,
    out_type=jax.ShapeDtypeStruct((2, NP, D), jnp.float32),
    mesh=_mesh,
    scratch_types=[
        pltpu.VMEM((BLK, CH), jnp.int32),
        pltpu.VMEM((BLK, CH), jnp.int32),
        pltpu.VMEM((BLK, CH), jnp.int32),
        pltpu.VMEM((BLK, CH), jnp.int32),
        pltpu.VMEM((CH, D), jnp.float32),
        pltpu.VMEM((CH, D), jnp.float32),
        pltpu.VMEM_SHARED((NP, D), jnp.float32),
        pltpu.SemaphoreType.DMA,
        pltpu.SemaphoreType.DMA,
        pltpu.SemaphoreType.DMA,
        pltpu.SemaphoreType.DMA,
        pltpu.SemaphoreType.DMA,
    ],
)


def _sc_deg_body(dst_hbm, out_hbm, dstv, zv, ov, acc):
    c = lax.axis_index("c")
    s = lax.axis_index("s")
    w = c * NSUB + s
    pltpu.sync_copy(dst_hbm.at[w], dstv)
    for k in range(CH // 16):
        zv[pl.ds(k * 16, 16)] = jnp.zeros((16,), jnp.float32)
        ov[pl.ds(k * 16, 16)] = jnp.ones((16,), jnp.float32)
    for k in range(STRIPE // CH):
        pltpu.sync_copy(zv, acc.at[pl.ds(s * STRIPE + k * CH, CH)])
    plsc.subcore_barrier()

    def chunk(i, carry):
        pltpu.sync_copy(ov, acc.at[dstv.at[i]], add=True)
        return carry

    lax.fori_loop(0, NCHUNK, chunk, 0)
    plsc.subcore_barrier()
    for k in range(STRIPE // CH):
        pltpu.sync_copy(acc.at[pl.ds(s * STRIPE + k * CH, CH)],
                        out_hbm.at[c, pl.ds(s * STRIPE + k * CH, CH)])


_sc_deg = pl.kernel(
    _sc_deg_body,
    out_type=jax.ShapeDtypeStruct((2, NP), jnp.float32),
    mesh=_mesh,
    scratch_types=[
        pltpu.VMEM((NCHUNK, CH), jnp.int32),
        pltpu.VMEM((CH,), jnp.float32),
        pltpu.VMEM((CH,), jnp.float32),
        pltpu.VMEM_SHARED((NP,), jnp.float32),
    ],
)


def kernel(x, edge_index, mask_vector, enc_token, dec_token,
           enc0_W, enc0_b, enc0_g, enc0_beta, enc0_a,
           enc1_W, enc1_b, enc1_g, enc1_beta, enc1_a,
           dec0_W, dec0_b, dec0_g, dec0_beta, dec0_a,
           dec1_W, dec1_b, dec1_g, dec1_beta, dec1_a,
           proj_W, proj_b):
    src = edge_index[0]
    dst = edge_index[1]
    pidx = jnp.arange(PAD, dtype=jnp.int32)
    srcp = jnp.concatenate([src, pidx % N]).reshape(NW, NCHUNK, CH)
    dstp = jnp.concatenate([dst, N + (pidx % 16)]).reshape(NW, NCHUNK, CH)

    degp = _sc_deg(dstp)
    deg = degp[0, :N] + degp[1, :N] + 1.0
    dinv = jax.lax.rsqrt(deg)[:, None]

    msk = (mask_vector == 0).astype(jnp.float32)[:, None]
    zero2d = jnp.zeros((NP, D), jnp.float32)

    def scatter(y):
        return _sc_scatter(y, srcp, dstp, zero2d)

    def r2(v):
        return v.reshape(1, -1)

    xin, y0 = _stage_in(x, msk, enc_token, dinv, enc0_W)
    s0 = scatter(y0)
    x1, y1 = _stage_mid(s0, y0, xin, dinv, r2(enc0_b), r2(enc0_g),
                        r2(enc0_beta), enc0_a.reshape(1, 1), enc1_W)
    s1 = scatter(y1)
    x2, y2 = _stage_mid_mask(s1, y1, x1, dinv, r2(enc1_b), r2(enc1_g),
                             r2(enc1_beta), enc1_a.reshape(1, 1), msk,
                             dec_token, dec0_W)
    s2 = scatter(y2)
    x3, y3 = _stage_mid(s2, y2, x2, dinv, r2(dec0_b), r2(dec0_g),
                        r2(dec0_beta), dec0_a.reshape(1, 1), dec1_W)
    s3 = scatter(y3)
    out = _stage_final(s3, y3, x3, dinv, r2(dec1_b), r2(dec1_g),
                       r2(dec1_beta), dec1_a.reshape(1, 1), proj_W,
                       r2(proj_b))
    return out
